# baseline (speedup 1.0000x reference)
import functools

import jax
import jax.numpy as jnp
from jax import lax
from jax.experimental import pallas as pl
from jax.experimental.pallas import tpu as pltpu
from jax.experimental.pallas import tpu_sc as plsc

NC = 2
NS = 16
NW = NC * NS

V = 1_000_000
D = 64
NB = 16384
NA = 26
NQ = V // 2

FULL_T = V // 128
TAIL_ROWS = V - FULL_T * 128
K1_ITERS = (FULL_T + NW - 1) // NW

OUT_BLOCKS = NA * NB // 128
K2_ITERS = OUT_BLOCKS // NW

_mesh = plsc.VectorSubcoreMesh(
    core_axis_name="c", subcore_axis_name="s", num_cores=NC, num_subcores=NS
)


def _wid():
    return lax.axis_index("s") * NC + lax.axis_index("c")


@functools.partial(
    pl.kernel,
    out_type=jax.ShapeDtypeStruct((NQ, 128), jnp.float32),
    mesh=_mesh,
    scratch_types=[
        pltpu.VMEM((64, 128), jnp.float32),
        pltpu.VMEM((64, 128), jnp.float32),
    ],
    compiler_params=pltpu.CompilerParams(needs_layout_passes=False),
)
def _pack_kernel(tT_hbm, tail_hbm, s_hbm, buf, sbuf):
    w = _wid()
    iota = lax.iota(jnp.int32, 16)

    def shuffle(m, nj):
        for g in range(8):
            cvec = (g % 4) * 16 + iota
            lvec = jnp.full((16,), 2 * m + g // 4, dtype=jnp.int32)
            vals = plsc.load_gather(buf, [cvec, lvec])
            sbuf[m, pl.ds(g * 16, 16)] = vals
        return nj

    def body(kb, carry):
        t = kb * NW + w

        @pl.when(t < FULL_T)
        def _():
            pltpu.sync_copy(tT_hbm.at[:, pl.ds(t * 128, 128)], buf)
            lax.fori_loop(0, 64, shuffle, 0, unroll=True)
            pltpu.sync_copy(sbuf, s_hbm.at[pl.ds(t * 64, 64)])

        return carry

    lax.fori_loop(0, K1_ITERS, body, 0)

    @pl.when(w == NW - 1)
    def _tail():
        pltpu.sync_copy(tail_hbm, sbuf.at[:32])
        pltpu.sync_copy(sbuf.at[:32], s_hbm.at[pl.ds(FULL_T * 64, 32)])


@functools.partial(
    pl.kernel,
    out_type=jax.ShapeDtypeStruct((NA, D, NB), jnp.float32),
    mesh=_mesh,
    scratch_types=[
        pltpu.VMEM((128,), jnp.int32),
        pltpu.VMEM((128,), jnp.int32),
        pltpu.VMEM((128, 128), jnp.float32),
        pltpu.VMEM((64, 128), jnp.float32),
        pltpu.SemaphoreType.DMA,
    ],
    compiler_params=pltpu.CompilerParams(needs_layout_passes=False),
)
def _gather_kernel(iT_hbm, s_hbm, out_hbm, idxb, qb, g2d, ob, sem):
    w = _wid()
    iota = lax.iota(jnp.int32, 16)

    def body(kb, carry):
        blk = kb * NW + w
        a = blk // 128
        b0 = (blk % 128) * 128

        pltpu.sync_copy(iT_hbm.at[a, pl.ds(b0, 128)], idxb)
        for g in range(8):
            qb[pl.ds(g * 16, 16)] = (
                jnp.right_shift(idxb[pl.ds(g * 16, 16)], 1)
            )
        pltpu.async_copy(s_hbm.at[qb], g2d, sem).wait()

        for g in range(8):
            rowvec = g * 16 + iota
            hoff = jnp.left_shift(
                jnp.bitwise_and(idxb[pl.ds(g * 16, 16)], 1), 6
            )
            for c in range(64):
                vals = plsc.load_gather(g2d, [rowvec, hoff + c])
                ob[c, pl.ds(g * 16, 16)] = vals

        pltpu.sync_copy(ob, out_hbm.at[a, :, pl.ds(b0, 128)])
        return carry

    lax.fori_loop(0, K2_ITERS, body, 0)


def kernel(i, types):
    tail = types[FULL_T * 128:].reshape(32, 128)
    s = _pack_kernel(types.T, tail)
    o3 = _gather_kernel(i.T, s)
    return o3.transpose(2, 0, 1)


# speedup vs baseline: 2.2837x; 2.2837x over previous
import functools

import jax
import jax.numpy as jnp
from jax import lax
from jax.experimental import pallas as pl
from jax.experimental.pallas import tpu as pltpu
from jax.experimental.pallas import tpu_sc as plsc

NC = 2
NS = 16
NW = NC * NS

V = 1_000_000
D = 64
NB = 16384
NA = 26
NQ = V // 2

FULL_T = V // 128
TAIL_ROWS = V - FULL_T * 128
K1_ITERS = (FULL_T + NW - 1) // NW

OUT_BLOCKS = NA * NB // 128
K2_ITERS = OUT_BLOCKS // NW

_mesh = plsc.VectorSubcoreMesh(
    core_axis_name="c", subcore_axis_name="s", num_cores=NC, num_subcores=NS
)


def _wid():
    return lax.axis_index("s") * NC + lax.axis_index("c")


@functools.partial(
    pl.kernel,
    out_type=jax.ShapeDtypeStruct((NQ, 128), jnp.float32),
    mesh=_mesh,
    scratch_types=[
        pltpu.VMEM((64, 128), jnp.float32),
        pltpu.VMEM((64, 128), jnp.float32),
        pltpu.VMEM((64, 128), jnp.float32),
        pltpu.VMEM((64, 128), jnp.float32),
        pltpu.SemaphoreType.DMA,
        pltpu.SemaphoreType.DMA,
        pltpu.SemaphoreType.DMA,
        pltpu.SemaphoreType.DMA,
    ],
    compiler_params=pltpu.CompilerParams(needs_layout_passes=False),
)
def _pack_kernel(tT_hbm, tail_hbm, s_hbm, b0_, b1_, s0_, s1_,
                 in0, in1, os0, os1):
    w = _wid()
    iota = lax.iota(jnp.int32, 16)
    bufs, sbufs = (b0_, b1_), (s0_, s1_)
    insems, osems = (in0, in1), (os0, os1)
    cvecs = [(g % 4) * 16 + iota for g in range(8)]

    def issue_in(kb, p):
        t = kb * NW + w

        @pl.when((kb < K1_ITERS) & (t < FULL_T))
        def _():
            pltpu.async_copy(
                tT_hbm.at[:, pl.ds(t * 128, 128)], bufs[p], insems[p]
            )

    def step(kb, p):
        t = kb * NW + w
        issue_in(kb + 1, 1 - p)

        @pl.when((kb < K1_ITERS) & (t < FULL_T))
        def _():
            pltpu.make_async_copy(
                tT_hbm.at[:, pl.ds(0, 128)], bufs[p], insems[p]
            ).wait()

            @pl.when(kb >= 2)
            def _w():
                pltpu.make_async_copy(
                    sbufs[p], s_hbm.at[pl.ds(0, 64)], osems[p]
                ).wait()

            buf, sbuf = bufs[p], sbufs[p]

            @plsc.parallel_loop(0, 64, unroll=2)
            def _sh(m):
                for g in range(8):
                    lvec = jnp.full((16,), 2 * m + g // 4, dtype=jnp.int32)
                    sbuf[m, pl.ds(g * 16, 16)] = plsc.load_gather(
                        buf, [cvecs[g], lvec]
                    )

            pltpu.async_copy(sbuf, s_hbm.at[pl.ds(t * 64, 64)], osems[p])

    issue_in(0, 0)

    def body(j, carry):
        step(2 * j, 0)
        step(2 * j + 1, 1)
        return carry

    lax.fori_loop(0, (K1_ITERS + 1) // 2, body, 0)

    for kb in (K1_ITERS - 2, K1_ITERS - 1):
        @pl.when(kb * NW + w < FULL_T)
        def _d():
            pltpu.make_async_copy(
                sbufs[kb % 2], s_hbm.at[pl.ds(0, 64)], osems[kb % 2]
            ).wait()

    @pl.when(w == NW - 1)
    def _tail():
        pltpu.sync_copy(tail_hbm, s0_.at[:32])
        pltpu.sync_copy(s0_.at[:32], s_hbm.at[pl.ds(FULL_T * 64, 32)])


@functools.partial(
    pl.kernel,
    out_type=jax.ShapeDtypeStruct((NA, D, NB), jnp.float32),
    mesh=_mesh,
    scratch_types=[
        pltpu.VMEM((128,), jnp.int32),
        pltpu.VMEM((128,), jnp.int32),
        pltpu.VMEM((128,), jnp.int32),
        pltpu.VMEM((128,), jnp.int32),
        pltpu.VMEM((128, 128), jnp.float32),
        pltpu.VMEM((128, 128), jnp.float32),
        pltpu.VMEM((64, 128), jnp.float32),
        pltpu.VMEM((64, 128), jnp.float32),
        pltpu.SemaphoreType.DMA,
        pltpu.SemaphoreType.DMA,
        pltpu.SemaphoreType.DMA,
        pltpu.SemaphoreType.DMA,
        pltpu.SemaphoreType.DMA,
        pltpu.SemaphoreType.DMA,
    ],
    compiler_params=pltpu.CompilerParams(needs_layout_passes=False),
)
def _gather_kernel(iT_hbm, s_hbm, out_hbm, ix0, ix1, q0_, q1_, g0_, g1_,
                   ob0, ob1, is0, is1, gs0, gs1, os0, os1):
    w = _wid()
    iota = lax.iota(jnp.int32, 16)
    idxbs, qbs, g2ds, obs = (ix0, ix1), (q0_, q1_), (g0_, g1_), (ob0, ob1)
    isems, gsems, osems = (is0, is1), (gs0, gs1), (os0, os1)
    rowvecs = [g * 16 + iota for g in range(8)]

    def blk_addr(kb):
        blk = kb * NW + w
        return blk // 128, (blk % 128) * 128

    def issue_idx(kb, p):
        @pl.when(kb < K2_ITERS)
        def _():
            a, b0 = blk_addr(kb)
            pltpu.async_copy(iT_hbm.at[a, pl.ds(b0, 128)], idxbs[p], isems[p])

    def launch_gather(kb, p):
        pltpu.make_async_copy(
            iT_hbm.at[0, pl.ds(0, 128)], idxbs[p], isems[p]
        ).wait()
        for g in range(8):
            qbs[p][pl.ds(g * 16, 16)] = jnp.right_shift(
                idxbs[p][pl.ds(g * 16, 16)], 1
            )
        pltpu.async_copy(s_hbm.at[qbs[p]], g2ds[p], gsems[p])

    def step(kb, p):
        a, b0 = blk_addr(kb)
        pltpu.make_async_copy(
            s_hbm.at[pl.ds(0, 128)], g2ds[p], gsems[p]
        ).wait()

        @pl.when(kb + 1 < K2_ITERS)
        def _():
            launch_gather(kb + 1, 1 - p)

        g2d, ob, idxb = g2ds[p], obs[p], idxbs[p]
        hoffs = [
            jnp.left_shift(
                jnp.bitwise_and(idxb[pl.ds(g * 16, 16)], 1), 6
            )
            for g in range(8)
        ]

        @pl.when(kb >= 2)
        def _w():
            pltpu.make_async_copy(
                obs[p], out_hbm.at[0, :, pl.ds(0, 128)], osems[p]
            ).wait()

        @plsc.parallel_loop(0, 64, unroll=2)
        def _sh(c):
            for g in range(8):
                ob[c, pl.ds(g * 16, 16)] = plsc.load_gather(
                    g2d, [rowvecs[g], hoffs[g] + c]
                )

        pltpu.async_copy(ob, out_hbm.at[a, :, pl.ds(b0, 128)], osems[p])
        issue_idx(kb + 2, p)

    issue_idx(0, 0)
    launch_gather(0, 0)
    issue_idx(1, 1)

    def body(j, carry):
        step(2 * j, 0)
        step(2 * j + 1, 1)
        return carry

    lax.fori_loop(0, K2_ITERS // 2, body, 0)

    for p in (0, 1):
        pltpu.make_async_copy(
            obs[p], out_hbm.at[0, :, pl.ds(0, 128)], osems[p]
        ).wait()


def kernel(i, types):
    tail = types[FULL_T * 128:].reshape(32, 128)
    s = _pack_kernel(types.T, tail)
    o3 = _gather_kernel(i.T, s)
    return o3.transpose(2, 0, 1)
